# baseline (speedup 1.0000x reference)
import functools

import jax
import jax.numpy as jnp
from jax.experimental import pallas as pl
from jax.experimental.pallas import tpu as pltpu


def _fused_body(bm, adj_ref, x16_ref, w0_ref, w1_ref, out_ref,
                z_scr, sx_scr, zn_scr, sxn_scr, deg_scr):
    l = pl.program_id(0)
    i = pl.program_id(1)
    dh = z_scr.shape[1]

    @pl.when((l == 0) & (i == 0))
    def _():
        z_scr[...] = jnp.dot(x16_ref[...], w0_ref[:, :dh],
                             preferred_element_type=jnp.float32
                             ).astype(jnp.bfloat16)
        sx_scr[...] = jnp.dot(x16_ref[...], w0_ref[:, dh:],
                              preferred_element_type=jnp.float32
                              ).astype(jnp.bfloat16)

    @pl.when((l == 1) & (i == 0))
    def _():
        z_scr[...] = zn_scr[...]
        sx_scr[...] = sxn_scr[...]

    a = adj_ref[...]
    p = jnp.dot(a.astype(jnp.bfloat16), z_scr[...],
                preferred_element_type=jnp.float32)

    @pl.when(l == 0)
    def _():
        deg_scr[pl.ds(i * bm, bm), :] = jnp.sum(a, axis=1, keepdims=True) + 1.0

    deg = deg_scr[pl.ds(i * bm, bm), :]
    out = sx_scr[pl.ds(i * bm, bm), :].astype(jnp.float32) + p * (1.0 / deg)
    out = jnp.where(l == 0, jnp.maximum(out, 0.0), out)

    @pl.when(l == 0)
    def _():
        h_blk = out.astype(jnp.bfloat16)
        zsx = jnp.dot(h_blk, w1_ref[...], preferred_element_type=jnp.float32
                      ).astype(jnp.bfloat16)
        zn_scr[pl.ds(i * bm, bm), :] = zsx[:, :dh]
        sxn_scr[pl.ds(i * bm, bm), :] = zsx[:, dh:]

    out_ref[...] = out


def kernel(adj, inputs, W0, W1):
    n, d_in = inputs.shape
    dh = W0.shape[0]
    bm = 1024
    x16 = inputs.astype(jnp.bfloat16)
    w0 = jnp.concatenate([W0[:, d_in:].T, W0[:, :d_in].T],
                         axis=1).astype(jnp.bfloat16)
    w1 = jnp.concatenate([W1[:, dh:].T, W1[:, :dh].T],
                         axis=1).astype(jnp.bfloat16)
    return pl.pallas_call(
        functools.partial(_fused_body, bm),
        grid=(2, n // bm),
        in_specs=[
            pl.BlockSpec((bm, n), lambda l, i: (i, 0)),
            pl.BlockSpec((n, d_in), lambda l, i: (0, 0)),
            pl.BlockSpec((d_in, 2 * dh), lambda l, i: (0, 0)),
            pl.BlockSpec((dh, 2 * dh), lambda l, i: (0, 0)),
        ],
        out_specs=pl.BlockSpec((bm, dh), lambda l, i: (l * i, 0)),
        out_shape=jax.ShapeDtypeStruct((n, dh), jnp.float32),
        scratch_shapes=[
            pltpu.VMEM((n, dh), jnp.bfloat16),
            pltpu.VMEM((n, dh), jnp.bfloat16),
            pltpu.VMEM((n, dh), jnp.bfloat16),
            pltpu.VMEM((n, dh), jnp.bfloat16),
            pltpu.VMEM((n, 1), jnp.float32),
        ],
        compiler_params=pltpu.CompilerParams(
            vmem_limit_bytes=67000000),
    )(adj, x16, w0, w1)


# speedup vs baseline: 1.0054x; 1.0054x over previous
import functools

import jax
import jax.numpy as jnp
from jax.experimental import pallas as pl
from jax.experimental.pallas import tpu as pltpu


def _fused_body(bm, adj_ref, x16_ref, wa0_ref, wb0_ref, wa1_ref,
                wb1_ref, out_ref, z_scr, sx_scr, h_scr, deg_scr):
    l = pl.program_id(0)
    i = pl.program_id(1)

    @pl.when((l == 0) & (i == 0))
    def _():
        z_scr[...] = jnp.dot(x16_ref[...], wb0_ref[...],
                             preferred_element_type=jnp.float32
                             ).astype(jnp.bfloat16)
        sx_scr[...] = jnp.dot(x16_ref[...], wa0_ref[...],
                              preferred_element_type=jnp.float32
                              ).astype(jnp.bfloat16)

    @pl.when((l == 1) & (i == 0))
    def _():
        z_scr[...] = jnp.dot(h_scr[...], wb1_ref[...],
                             preferred_element_type=jnp.float32
                             ).astype(jnp.bfloat16)
        sx_scr[...] = jnp.dot(h_scr[...], wa1_ref[...],
                              preferred_element_type=jnp.float32
                              ).astype(jnp.bfloat16)

    a16 = adj_ref[...].astype(jnp.bfloat16)
    p = jnp.dot(a16, z_scr[...], preferred_element_type=jnp.float32)

    @pl.when(l == 0)
    def _():
        deg_scr[pl.ds(i * bm, bm), :] = (
            jnp.sum(a16.astype(jnp.float32), axis=1, keepdims=True) + 1.0)

    deg = deg_scr[pl.ds(i * bm, bm), :]
    out = sx_scr[pl.ds(i * bm, bm), :].astype(jnp.float32) + p * (1.0 / deg)
    out = jnp.where(l == 0, jnp.maximum(out, 0.0), out)

    @pl.when(l == 0)
    def _():
        h_scr[pl.ds(i * bm, bm), :] = out.astype(jnp.bfloat16)

    out_ref[...] = out


def kernel(adj, inputs, W0, W1):
    n, d_in = inputs.shape
    dh = W0.shape[0]
    bm = 1024
    x16 = inputs.astype(jnp.bfloat16)
    wa0, wb0 = (W0[:, :d_in].T.astype(jnp.bfloat16),
                W0[:, d_in:].T.astype(jnp.bfloat16))
    wa1, wb1 = (W1[:, :dh].T.astype(jnp.bfloat16),
                W1[:, dh:].T.astype(jnp.bfloat16))
    return pl.pallas_call(
        functools.partial(_fused_body, bm),
        grid=(2, n // bm),
        in_specs=[
            pl.BlockSpec((bm, n), lambda l, i: (i, 0)),
            pl.BlockSpec((n, d_in), lambda l, i: (0, 0)),
            pl.BlockSpec((d_in, dh), lambda l, i: (0, 0)),
            pl.BlockSpec((d_in, dh), lambda l, i: (0, 0)),
            pl.BlockSpec((dh, dh), lambda l, i: (0, 0)),
            pl.BlockSpec((dh, dh), lambda l, i: (0, 0)),
        ],
        out_specs=pl.BlockSpec((bm, dh), lambda l, i: (l * i, 0)),
        out_shape=jax.ShapeDtypeStruct((n, dh), jnp.float32),
        scratch_shapes=[
            pltpu.VMEM((n, dh), jnp.bfloat16),
            pltpu.VMEM((n, dh), jnp.bfloat16),
            pltpu.VMEM((n, dh), jnp.bfloat16),
            pltpu.VMEM((n, 1), jnp.float32),
        ],
        compiler_params=pltpu.CompilerParams(
            vmem_limit_bytes=100 * 1024 * 1024),
    )(adj, x16, wa0, wb0, wa1, wb1)


# speedup vs baseline: 1.0139x; 1.0084x over previous
import functools

import jax
import jax.numpy as jnp
from jax.experimental import pallas as pl
from jax.experimental.pallas import tpu as pltpu


def _fused_body(bm, adj_ref, x16_ref, wa0_ref, wb0_ref, wa1_ref,
                wb1_ref, out_ref, z_scr, sx_scr, h_scr, deg_scr):
    l = pl.program_id(0)
    i = pl.program_id(1)

    @pl.when((l == 0) & (i == 0))
    def _():
        z_scr[...] = jnp.dot(x16_ref[...], wb0_ref[...],
                             preferred_element_type=jnp.float32
                             ).astype(jnp.bfloat16)
        sx_scr[...] = jnp.dot(x16_ref[...], wa0_ref[...],
                              preferred_element_type=jnp.float32
                              ).astype(jnp.bfloat16)

    @pl.when((l == 1) & (i == 0))
    def _():
        z_scr[...] = jnp.dot(h_scr[...], wb1_ref[...],
                             preferred_element_type=jnp.float32
                             ).astype(jnp.bfloat16)
        sx_scr[...] = jnp.dot(h_scr[...], wa1_ref[...],
                              preferred_element_type=jnp.float32
                              ).astype(jnp.bfloat16)

    a = adj_ref[...]
    p = jnp.dot(a.astype(jnp.bfloat16), z_scr[...],
                preferred_element_type=jnp.float32)

    @pl.when(l == 0)
    def _():
        deg_scr[pl.ds(i * bm, bm), :] = jnp.sum(a, axis=1, keepdims=True) + 1.0

    deg = deg_scr[pl.ds(i * bm, bm), :]
    out = sx_scr[pl.ds(i * bm, bm), :].astype(jnp.float32) + p * (1.0 / deg)
    out = jnp.where(l == 0, jnp.maximum(out, 0.0), out)

    @pl.when(l == 0)
    def _():
        h_scr[pl.ds(i * bm, bm), :] = out.astype(jnp.bfloat16)

    out_ref[...] = out


def kernel(adj, inputs, W0, W1):
    n, d_in = inputs.shape
    dh = W0.shape[0]
    bm = 1024
    x16 = inputs.astype(jnp.bfloat16)
    wa0, wb0 = (W0[:, :d_in].T.astype(jnp.bfloat16),
                W0[:, d_in:].T.astype(jnp.bfloat16))
    wa1, wb1 = (W1[:, :dh].T.astype(jnp.bfloat16),
                W1[:, dh:].T.astype(jnp.bfloat16))
    return pl.pallas_call(
        functools.partial(_fused_body, bm),
        grid=(2, n // bm),
        in_specs=[
            pl.BlockSpec((bm, n), lambda l, i: (i, 0)),
            pl.BlockSpec((n, d_in), lambda l, i: (0, 0)),
            pl.BlockSpec((d_in, dh), lambda l, i: (0, 0)),
            pl.BlockSpec((d_in, dh), lambda l, i: (0, 0)),
            pl.BlockSpec((dh, dh), lambda l, i: (0, 0)),
            pl.BlockSpec((dh, dh), lambda l, i: (0, 0)),
        ],
        out_specs=pl.BlockSpec((bm, dh), lambda l, i: (l * i, 0)),
        out_shape=jax.ShapeDtypeStruct((n, dh), jnp.float32),
        scratch_shapes=[
            pltpu.VMEM((n, dh), jnp.bfloat16),
            pltpu.VMEM((n, dh), jnp.bfloat16),
            pltpu.VMEM((n, dh), jnp.bfloat16),
            pltpu.VMEM((n, 1), jnp.float32),
        ],
        compiler_params=pltpu.CompilerParams(
            vmem_limit_bytes=100 * 1024 * 1024),
    )(adj, x16, wa0, wb0, wa1, wb1)
